# Initial kernel scaffold; baseline (speedup 1.0000x reference)
import jax
import jax.numpy as jnp
from jax.experimental import pallas as pl


def kernel(logits, scores):
    raise NotImplementedError("write your pallas kernel here")


# speedup vs baseline: 9.4629x; 9.4629x over previous
import functools

import jax
import jax.numpy as jnp
from jax import lax
from jax.experimental import pallas as pl
from jax.experimental.pallas import tpu as pltpu
from jax.experimental.pallas import tpu_sc as plsc

V = 100000
BATCH = 64
BEAM = 4
K = 8
NSW = 9
CW = 2048
NCHUNK = 24
REMW = 1664
SLABC_W = 1792
GMB = 6400
NEG = -1e30
IMAX = 2**31 - 1
LN2 = 0.6931471805599453


def _log_f32(v):
    bits = lax.bitcast_convert_type(v, jnp.int32)
    e = ((bits >> 23) & 0xFF) - 127
    mant = lax.bitcast_convert_type((bits & 0x7FFFFF) | 0x3F800000,
                                    jnp.float32)
    big = mant > 1.41421356
    mant = jnp.where(big, mant * 0.5, mant)
    ef = jnp.where(big, e + 1, e).astype(jnp.float32)
    t = (mant - 1.0) / (mant + 1.0)
    t2 = t * t
    p = 2.0 * t * (1.0 + t2 * (1.0 / 3.0 + t2 * (0.2 + t2 * (1.0 / 7.0))))
    return ef * LN2 + p


@functools.partial(
    pl.kernel,
    mesh=plsc.VectorSubcoreMesh(core_axis_name="c", subcore_axis_name="s"),
    compiler_params=pltpu.CompilerParams(needs_layout_passes=False),
    out_type=[
        jax.ShapeDtypeStruct((BATCH * 2 * BEAM,), jnp.float32),
        jax.ShapeDtypeStruct((BATCH * 2 * BEAM,), jnp.int32),
        jax.ShapeDtypeStruct((BATCH * 2 * BEAM,), jnp.int32),
    ],
    scratch_types=[
        pltpu.VMEM((BEAM, CW), jnp.float32),
        pltpu.VMEM((BEAM, CW), jnp.float32),
        pltpu.VMEM((BEAM, SLABC_W), jnp.float32),
        pltpu.VMEM((BEAM * 32,), jnp.float32),
        pltpu.VMEM((BEAM * GMB,), jnp.float32),
        pltpu.VMEM((400,), jnp.float32),
        pltpu.VMEM((BEAM * 32,), jnp.float32),
        pltpu.VMEM((16,), jnp.float32),
        pltpu.VMEM((BEAM * 16,), jnp.float32),
        pltpu.VMEM((BEAM * 16,), jnp.int32),
        pltpu.VMEM((BEAM * 16,), jnp.float32),
        pltpu.VMEM((NSW, BEAM, 256), jnp.float32),
        pltpu.VMEM((16,), jnp.float32),
        pltpu.VMEM((16,), jnp.int32),
        pltpu.VMEM((16,), jnp.int32),
        pltpu.SemaphoreType.DMA,
        pltpu.SemaphoreType.DMA,
        pltpu.SemaphoreType.DMA,
        pltpu.SemaphoreType.DMA,
    ],
)
def _topk_sc(logits_hbm, tail_hbm, scores_hbm, vals_hbm, beam_hbm, tok_hbm,
             slaba, slabb, slabc, tailbuf, gmbuf, l2buf, msbuf, scorebuf,
             resvals, residx, stats, candbuf, stagef, stageb, staget,
             sema, semb, semr, semc):
    tid = lax.axis_index("s") * 2 + lax.axis_index("c")
    lane = lax.iota(jnp.int32, 16)
    negv = jnp.full((16,), NEG, jnp.float32)
    zf = jnp.zeros((16,), jnp.float32)
    zi = jnp.zeros((16,), jnp.int32)

    for m in range(BEAM):
        for k in range((GMB - 6256) // 16):
            gmbuf[pl.ds(m * GMB + 6256 + k * 16, 16)] = negv
        for k in range((SLABC_W - 1696) // 16):
            slabc[m, pl.ds(1696 + k * 16, 16)] = negv
    scorebuf[...] = zf
    pltpu.sync_copy(scores_hbm.at[pl.ds(tid * 8, 8)], scorebuf.at[pl.ds(0, 8)])

    def stream_pairs(src, p, pbase, ms):
        out = []
        base = pl.multiple_of(p * 256, 256)
        for m in range(BEAM):
            mv, sv = ms[m]
            vs = [src[m, pl.ds(base + k * 16, 16)] for k in range(16)]
            t = vs
            for stride in (8, 4, 2, 1):
                t = [jnp.maximum(t[j], t[j + stride]) for j in range(stride)]
            pm = t[0]
            gmbuf[pl.ds(m * GMB + (pbase + p) * 16, 16)] = pm
            newm = jnp.maximum(mv, pm)
            sv = sv * jnp.exp(mv - newm)
            for k in range(16):
                sv = sv + jnp.exp(vs[k] - newm)
            out.append((newm, sv))
        return out

    def batch_body(bl, _carry):
        b = tid * 2 + bl
        pltpu.async_copy(logits_hbm.at[b, :, pl.ds(0, CW)], slaba, sema)
        pltpu.async_copy(logits_hbm.at[b, :, pl.ds(48 * CW, REMW)],
                         slabc.at[:, pl.ds(0, REMW)], semr)
        pltpu.async_copy(tail_hbm.at[pl.ds(b * 128, 128)], tailbuf, semr)

        def chunk_body(i, ms8):
            c0 = pl.multiple_of((2 * i + 1) * CW, CW)
            pltpu.async_copy(logits_hbm.at[b, :, pl.ds(c0, CW)], slabb, semb)
            pltpu.make_async_copy(logits_hbm.at[b, :, pl.ds(0, CW)],
                                  slaba, sema).wait()

            def pair_a(p, ms8i):
                msi = [(ms8i[2 * m], ms8i[2 * m + 1]) for m in range(BEAM)]
                r = stream_pairs(slaba, p, i * 16, msi)
                return tuple(x for mv_sv in r for x in mv_sv)

            ms8 = lax.fori_loop(0, 8, pair_a, ms8)

            @pl.when(i < NCHUNK - 1)
            def _():
                c1 = pl.multiple_of((2 * i + 2) * CW, CW)
                pltpu.async_copy(logits_hbm.at[b, :, pl.ds(c1, CW)],
                                 slaba, sema)

            pltpu.make_async_copy(logits_hbm.at[b, :, pl.ds(0, CW)],
                                  slabb, semb).wait()

            def pair_b(p, ms8i):
                msi = [(ms8i[2 * m], ms8i[2 * m + 1]) for m in range(BEAM)]
                r = stream_pairs(slabb, p, i * 16 + 8, msi)
                return tuple(x for mv_sv in r for x in mv_sv)

            return lax.fori_loop(0, 8, pair_b, ms8)

        init = tuple(x for _ in range(BEAM) for x in (negv, zf))
        ms8 = lax.fori_loop(0, NCHUNK, chunk_body, init)

        pltpu.make_async_copy(logits_hbm.at[b, :, pl.ds(0, REMW)],
                              slabc.at[:, pl.ds(0, REMW)], semr).wait()
        pltpu.make_async_copy(tail_hbm.at[pl.ds(0, 128)], tailbuf,
                              semr).wait()
        for m in range(BEAM):
            slabc[m, pl.ds(1664, 16)] = tailbuf[pl.ds(m * 32, 16)]
            slabc[m, pl.ds(1680, 16)] = tailbuf[pl.ds(m * 32 + 16, 16)]

        def pair_c(p, ms8i):
            msi = [(ms8i[2 * m], ms8i[2 * m + 1]) for m in range(BEAM)]
            r = stream_pairs(slabc, p, 384, msi)
            return tuple(x for mv_sv in r for x in mv_sv)

        ms8 = lax.fori_loop(0, 7, pair_c, ms8)
        for m in range(BEAM):
            msbuf[pl.ds(m * 32, 16)] = ms8[2 * m]
            msbuf[pl.ds(m * 32 + 16, 16)] = ms8[2 * m + 1]

        def beam_body(m, _c):
            moff = m * 32
            mv = msbuf[pl.ds(moff, 16)]
            sv = msbuf[pl.ds(moff + 16, 16)]
            mrow = jnp.max(mv)
            s = jnp.sum(sv * jnp.exp(mv - mrow))
            lse = mrow + jnp.max(_log_f32(jnp.full((16,), s, jnp.float32)))
            sc = jnp.max(jnp.where(lane == bl * BEAM + m, scorebuf[...], NEG))
            adj = sc - lse

            def l2_body(cl, _):
                base = pl.multiple_of(cl * 256, 256)
                t = [gmbuf[pl.ds(m * GMB + base + k * 16, 16)]
                     for k in range(16)]
                for stride in (8, 4, 2, 1):
                    t = [jnp.maximum(t[j], t[j + stride])
                         for j in range(stride)]
                l2buf[pl.ds(pl.multiple_of(cl * 16, 16), 16)] = t[0]
                return 0

            lax.fori_loop(0, 25, l2_body, 0)

            wv = zi
            for sw in range(NSW):
                val_v, g2v = negv, zi
                for cl in range(25):
                    cur = l2buf[pl.ds(cl * 16, 16)]
                    upd = cur > val_v
                    val_v = jnp.where(upd, cur, val_v)
                    g2v = jnp.where(upd, cl * 16 + lane, g2v)
                ml2 = jnp.max(val_v)
                g2 = jnp.min(jnp.where(val_v == ml2, g2v, IMAX))
                idxrel = (g2 >> 4) * 256 + (g2 & 15) + 16 * lane
                gmv = plsc.load_gather(gmbuf, [m * GMB + idxrel])
                g = jnp.min(jnp.where(gmv == ml2, idxrel, IMAX))
                plsc.store_scatter(gmbuf,
                                   [jnp.full((16,), m * GMB + g, jnp.int32)],
                                   negv, mask=lane == 0)
                newl2 = jnp.max(jnp.where(idxrel == g, NEG, gmv))
                plsc.store_scatter(l2buf, [jnp.full((16,), g2, jnp.int32)],
                                   jnp.full((16,), newl2, jnp.float32),
                                   mask=lane == 0)
                wv = jnp.where(lane == sw, g, wv)

            gs = []
            for w in range(NSW):
                g = jnp.max(jnp.where(lane == w, wv, -1))
                gs.append(g)
                p = g >> 4
                srccol = pl.multiple_of(jnp.where(p < 390, p, 0) * 256, 256)
                pltpu.async_copy(logits_hbm.at[b, :, pl.ds(srccol, 256)],
                                 candbuf.at[w], semc)
            for w in range(NSW):
                pltpu.make_async_copy(logits_hbm.at[b, :, pl.ds(0, 256)],
                                      candbuf.at[w], semc).wait()

            cand_v, cand_c = [], []
            mfull = jnp.full((16,), m, jnp.int32)
            for w in range(NSW):
                g = gs[w]
                p, l = g >> 4, g & 15
                colv = p * 256 + l + 16 * lane
                va = plsc.load_gather(
                    candbuf, [jnp.full((16,), w, jnp.int32), mfull,
                              l + 16 * lane])
                vb = plsc.load_gather(slabc, [mfull, 1536 + l + 16 * lane])
                val = jnp.where(p == 390, vb, va)
                val = jnp.where((colv >= V) | (colv == 1), NEG, val)
                cand_v.append(val)
                cand_c.append(colv)

            v8, i8 = negv, zi
            for j in range(K):
                mx = cand_v[0]
                for q in range(1, NSW):
                    mx = jnp.maximum(mx, cand_v[q])
                mc = jnp.max(mx)
                cur = jnp.full((16,), IMAX, jnp.int32)
                for q in range(NSW):
                    cur = jnp.minimum(cur, jnp.where(cand_v[q] == mc,
                                                     cand_c[q], IMAX))
                chosen = jnp.min(cur)
                v8 = jnp.where(lane == j, mc, v8)
                i8 = jnp.where(lane == j, chosen, i8)
                cand_v = [jnp.where(cand_c[q] == chosen, NEG, cand_v[q])
                          for q in range(NSW)]

            base16 = pl.multiple_of(m * 16, 16)
            resvals[pl.ds(base16, 16)] = v8
            residx[pl.ds(base16, 16)] = i8
            stats[pl.ds(base16, 16)] = jnp.where(lane == 0, adj, 0.0)
            return 0

        lax.fori_loop(0, BEAM, beam_body, 0)

        comb, flat = [], []
        for m in range(BEAM):
            svv = stats[pl.ds(m * 16, 16)]
            adj = jnp.max(jnp.where(lane == 0, svv, NEG))
            comb.append(jnp.where(lane < K,
                                  resvals[pl.ds(m * 16, 16)] + adj, NEG))
            flat.append(residx[pl.ds(m * 16, 16)] + m * V)
        ov, ob, ot = negv, zi, zi
        for j in range(K):
            mx = jnp.maximum(jnp.maximum(comb[0], comb[1]),
                             jnp.maximum(comb[2], comb[3]))
            mc = jnp.max(mx)
            cur = jnp.full((16,), IMAX, jnp.int32)
            for m in range(BEAM):
                cur = jnp.minimum(cur, jnp.where(comb[m] == mc,
                                                 flat[m], IMAX))
            fm = jnp.min(cur)
            ov = jnp.where(lane == j, mc, ov)
            ob = jnp.where(lane == j, fm // V, ob)
            ot = jnp.where(lane == j, fm % V, ot)
            comb = [jnp.where(flat[m] == fm, NEG, comb[m])
                    for m in range(BEAM)]
        stagef[...] = ov
        stageb[...] = ob
        staget[...] = ot
        obase = pl.multiple_of(b * K, 8)
        pltpu.sync_copy(stagef.at[pl.ds(0, K)], vals_hbm.at[pl.ds(obase, K)])
        pltpu.sync_copy(stageb.at[pl.ds(0, K)], beam_hbm.at[pl.ds(obase, K)])
        pltpu.sync_copy(staget.at[pl.ds(0, K)], tok_hbm.at[pl.ds(obase, K)])
        return 0

    lax.fori_loop(0, 2, batch_body, 0)


def kernel(logits, scores):
    tail = logits[:, :, V - 32:].reshape(BATCH * BEAM * 32)
    flat_scores = scores.reshape(BATCH * BEAM)
    vals, beams, toks = _topk_sc(logits, tail, flat_scores)
    shape = (BATCH, 2 * BEAM)
    return vals.reshape(shape), beams.reshape(shape), toks.reshape(shape)


# speedup vs baseline: 9.5728x; 1.0116x over previous
import functools

import jax
import jax.numpy as jnp
from jax import lax
from jax.experimental import pallas as pl
from jax.experimental.pallas import tpu as pltpu
from jax.experimental.pallas import tpu_sc as plsc

V = 100000
BATCH = 64
BEAM = 4
K = 8
NSW = 9
CW = 2048
NCHUNK = 24
REMW = 1664
SLABC_W = 1792
GMB = 6400
NEG = -1e30
IMAX = 2**31 - 1
LN2 = 0.6931471805599453


def _log_f32(v):
    bits = lax.bitcast_convert_type(v, jnp.int32)
    e = ((bits >> 23) & 0xFF) - 127
    mant = lax.bitcast_convert_type((bits & 0x7FFFFF) | 0x3F800000,
                                    jnp.float32)
    big = mant > 1.41421356
    mant = jnp.where(big, mant * 0.5, mant)
    ef = jnp.where(big, e + 1, e).astype(jnp.float32)
    t = (mant - 1.0) / (mant + 1.0)
    t2 = t * t
    p = 2.0 * t * (1.0 + t2 * (1.0 / 3.0 + t2 * (0.2 + t2 * (1.0 / 7.0))))
    return ef * LN2 + p


@functools.partial(
    pl.kernel,
    mesh=plsc.VectorSubcoreMesh(core_axis_name="c", subcore_axis_name="s"),
    compiler_params=pltpu.CompilerParams(needs_layout_passes=False),
    out_type=[
        jax.ShapeDtypeStruct((BATCH * 2 * BEAM,), jnp.float32),
        jax.ShapeDtypeStruct((BATCH * 2 * BEAM,), jnp.int32),
        jax.ShapeDtypeStruct((BATCH * 2 * BEAM,), jnp.int32),
    ],
    scratch_types=[
        pltpu.VMEM((BEAM, CW), jnp.float32),
        pltpu.VMEM((BEAM, CW), jnp.float32),
        pltpu.VMEM((BEAM, SLABC_W), jnp.float32),
        pltpu.VMEM((BEAM * 32,), jnp.float32),
        pltpu.VMEM((BEAM * GMB,), jnp.float32),
        pltpu.VMEM((BEAM * 16,), jnp.float32),
        pltpu.VMEM((16,), jnp.float32),
        pltpu.VMEM((BEAM * 16,), jnp.float32),
        pltpu.VMEM((BEAM * 16,), jnp.int32),
        pltpu.VMEM((BEAM * 16,), jnp.float32),
        pltpu.VMEM((BEAM * 16,), jnp.int32),
        pltpu.VMEM((NSW * BEAM, BEAM, 256), jnp.float32),
        pltpu.VMEM((16,), jnp.float32),
        pltpu.VMEM((16,), jnp.int32),
        pltpu.VMEM((16,), jnp.int32),
        pltpu.SemaphoreType.DMA,
        pltpu.SemaphoreType.DMA,
        pltpu.SemaphoreType.DMA,
        pltpu.SemaphoreType.DMA,
    ],
)
def _topk_sc(logits_hbm, tail_hbm, scores_hbm, vals_hbm, beam_hbm, tok_hbm,
             slaba, slabb, slabc, tailbuf, gmbuf, msbuf, scorebuf,
             resvals, residx, stats, winbuf, candbuf, stagef, stageb, staget,
             sema, semb, semr, semc):
    tid = lax.axis_index("s") * 2 + lax.axis_index("c")
    lane = lax.iota(jnp.int32, 16)
    negv = jnp.full((16,), NEG, jnp.float32)
    zf = jnp.zeros((16,), jnp.float32)
    zi = jnp.zeros((16,), jnp.int32)

    for m in range(BEAM):
        for k in range((GMB - 6256) // 16):
            gmbuf[pl.ds(m * GMB + 6256 + k * 16, 16)] = negv
        for k in range((SLABC_W - 1696) // 16):
            slabc[m, pl.ds(1696 + k * 16, 16)] = negv
    scorebuf[...] = zf
    pltpu.sync_copy(scores_hbm.at[pl.ds(tid * 8, 8)], scorebuf.at[pl.ds(0, 8)])

    def stream_pairs(src, p, pbase, svs):
        out = []
        base = pl.multiple_of(p * 256, 256)
        for m in range(BEAM):
            vs = [src[m, pl.ds(base + k * 16, 16)] for k in range(16)]
            t = vs
            for stride in (8, 4, 2, 1):
                t = [jnp.maximum(t[j], t[j + stride]) for j in range(stride)]
            gmbuf[pl.ds(m * GMB + (pbase + p) * 16, 16)] = t[0]
            es = [jnp.exp(v) for v in vs]
            for stride in (8, 4, 2, 1):
                es = [es[j] + es[j + stride] for j in range(stride)]
            out.append(svs[m] + es[0])
        return tuple(out)

    def batch_body(bl, _carry):
        b = tid * 2 + bl
        pltpu.async_copy(logits_hbm.at[b, :, pl.ds(0, CW)], slaba, sema)
        pltpu.async_copy(logits_hbm.at[b, :, pl.ds(48 * CW, REMW)],
                         slabc.at[:, pl.ds(0, REMW)], semr)
        pltpu.async_copy(tail_hbm.at[pl.ds(b * 128, 128)], tailbuf, semr)

        def chunk_body(i, svs):
            c0 = pl.multiple_of((2 * i + 1) * CW, CW)
            pltpu.async_copy(logits_hbm.at[b, :, pl.ds(c0, CW)], slabb, semb)
            pltpu.make_async_copy(logits_hbm.at[b, :, pl.ds(0, CW)],
                                  slaba, sema).wait()

            def pair_a(p, svsi):
                return stream_pairs(slaba, p, i * 16, svsi)

            svs = lax.fori_loop(0, 8, pair_a, svs)

            @pl.when(i < NCHUNK - 1)
            def _():
                c1 = pl.multiple_of((2 * i + 2) * CW, CW)
                pltpu.async_copy(logits_hbm.at[b, :, pl.ds(c1, CW)],
                                 slaba, sema)

            pltpu.make_async_copy(logits_hbm.at[b, :, pl.ds(0, CW)],
                                  slabb, semb).wait()

            def pair_b(p, svsi):
                return stream_pairs(slabb, p, i * 16 + 8, svsi)

            return lax.fori_loop(0, 8, pair_b, svs)

        svs = lax.fori_loop(0, NCHUNK, chunk_body, (zf, zf, zf, zf))

        pltpu.make_async_copy(logits_hbm.at[b, :, pl.ds(0, REMW)],
                              slabc.at[:, pl.ds(0, REMW)], semr).wait()
        pltpu.make_async_copy(tail_hbm.at[pl.ds(0, 128)], tailbuf,
                              semr).wait()
        for m in range(BEAM):
            slabc[m, pl.ds(1664, 16)] = tailbuf[pl.ds(m * 32, 16)]
            slabc[m, pl.ds(1680, 16)] = tailbuf[pl.ds(m * 32 + 16, 16)]

        def pair_c(p, svsi):
            return stream_pairs(slabc, p, 384, svsi)

        svs = lax.fori_loop(0, 7, pair_c, svs)
        for m in range(BEAM):
            msbuf[pl.ds(m * 16, 16)] = svs[m]

        def beam_sel(m, _c):
            sv = msbuf[pl.ds(m * 16, 16)]
            s = jnp.sum(sv)
            lse = jnp.max(_log_f32(jnp.full((16,), s, jnp.float32)))
            sc = jnp.max(jnp.where(lane == bl * BEAM + m, scorebuf[...], NEG))
            adj = sc - lse

            l2 = []
            for cl in range(25):
                t = [gmbuf[pl.ds(m * GMB + cl * 256 + k * 16, 16)]
                     for k in range(16)]
                for stride in (8, 4, 2, 1):
                    t = [jnp.maximum(t[j], t[j + stride])
                         for j in range(stride)]
                l2.append(t[0])

            wv = zi
            for sw in range(NSW):
                val_v, g2v = negv, zi
                for cl in range(25):
                    upd = l2[cl] > val_v
                    val_v = jnp.where(upd, l2[cl], val_v)
                    g2v = jnp.where(upd, cl * 16 + lane, g2v)
                ml2 = jnp.max(val_v)
                g2 = jnp.min(jnp.where(val_v == ml2, g2v, IMAX))
                clsel, l2l = g2 >> 4, g2 & 15
                idxrel = clsel * 256 + l2l + 16 * lane
                gmv = plsc.load_gather(gmbuf, [m * GMB + idxrel])
                g = jnp.min(jnp.where(gmv == ml2, idxrel, IMAX))
                plsc.store_scatter(gmbuf,
                                   [jnp.full((16,), m * GMB + g, jnp.int32)],
                                   negv, mask=lane == 0)
                newl2 = jnp.max(jnp.where(idxrel == g, NEG, gmv))
                hit = (lane == l2l) & jnp.full((16,), True)
                l2 = [jnp.where((cl == clsel) & hit, newl2, l2[cl])
                      for cl in range(25)]
                wv = jnp.where(lane == sw, g, wv)

                p = g >> 4
                srccol = pl.multiple_of(jnp.where(p < 390, p, 0) * 256, 256)
                pltpu.async_copy(logits_hbm.at[b, :, pl.ds(srccol, 256)],
                                 candbuf.at[m * NSW + sw], semc)

            winbuf[pl.ds(m * 16, 16)] = wv
            resvals[pl.ds(pl.multiple_of(m * 16, 16), 16)] = negv
            stats[pl.ds(m * 16, 16)] = jnp.where(lane == 0, adj, 0.0)
            return 0

        lax.fori_loop(0, BEAM, beam_sel, 0)
        for _w in range(NSW * BEAM):
            pltpu.make_async_copy(logits_hbm.at[b, :, pl.ds(0, 256)],
                                  candbuf.at[0], semc).wait()

        def beam_top(m, _c):
            wv = winbuf[pl.ds(m * 16, 16)]
            mfull = jnp.full((16,), m, jnp.int32)
            cand_v, cand_c = [], []
            for w in range(NSW):
                g = jnp.max(jnp.where(lane == w, wv, -1))
                p, l = g >> 4, g & 15
                colv = p * 256 + l + 16 * lane
                va = plsc.load_gather(
                    candbuf, [jnp.full((16,), m * NSW + w, jnp.int32), mfull,
                              l + 16 * lane])
                vb = plsc.load_gather(slabc, [mfull, 1536 + l + 16 * lane])
                val = jnp.where(p == 390, vb, va)
                val = jnp.where((colv >= V) | (colv == 1), NEG, val)
                cand_v.append(val)
                cand_c.append(colv)

            v8, i8 = negv, zi
            for j in range(K):
                mx = cand_v[0]
                for q in range(1, NSW):
                    mx = jnp.maximum(mx, cand_v[q])
                mc = jnp.max(mx)
                cur = jnp.full((16,), IMAX, jnp.int32)
                for q in range(NSW):
                    cur = jnp.minimum(cur, jnp.where(cand_v[q] == mc,
                                                     cand_c[q], IMAX))
                chosen = jnp.min(cur)
                v8 = jnp.where(lane == j, mc, v8)
                i8 = jnp.where(lane == j, chosen, i8)
                cand_v = [jnp.where(cand_c[q] == chosen, NEG, cand_v[q])
                          for q in range(NSW)]

            resvals[pl.ds(m * 16, 16)] = v8
            residx[pl.ds(m * 16, 16)] = i8
            return 0

        lax.fori_loop(0, BEAM, beam_top, 0)

        comb, flat = [], []
        for m in range(BEAM):
            svv = stats[pl.ds(m * 16, 16)]
            adj = jnp.max(jnp.where(lane == 0, svv, NEG))
            comb.append(jnp.where(lane < K,
                                  resvals[pl.ds(m * 16, 16)] + adj, NEG))
            flat.append(residx[pl.ds(m * 16, 16)] + m * V)
        ov, ob, ot = negv, zi, zi
        for j in range(K):
            mx = jnp.maximum(jnp.maximum(comb[0], comb[1]),
                             jnp.maximum(comb[2], comb[3]))
            mc = jnp.max(mx)
            cur = jnp.full((16,), IMAX, jnp.int32)
            for m in range(BEAM):
                cur = jnp.minimum(cur, jnp.where(comb[m] == mc,
                                                 flat[m], IMAX))
            fm = jnp.min(cur)
            ov = jnp.where(lane == j, mc, ov)
            ob = jnp.where(lane == j, fm // V, ob)
            ot = jnp.where(lane == j, fm % V, ot)
            comb = [jnp.where(flat[m] == fm, NEG, comb[m])
                    for m in range(BEAM)]
        stagef[...] = ov
        stageb[...] = ob
        staget[...] = ot
        obase = pl.multiple_of(b * K, 8)
        pltpu.sync_copy(stagef.at[pl.ds(0, K)], vals_hbm.at[pl.ds(obase, K)])
        pltpu.sync_copy(stageb.at[pl.ds(0, K)], beam_hbm.at[pl.ds(obase, K)])
        pltpu.sync_copy(staget.at[pl.ds(0, K)], tok_hbm.at[pl.ds(obase, K)])
        return 0

    lax.fori_loop(0, 2, batch_body, 0)


def kernel(logits, scores):
    tail = logits[:, :, V - 32:].reshape(BATCH * BEAM * 32)
    flat_scores = scores.reshape(BATCH * BEAM)
    vals, beams, toks = _topk_sc(logits, tail, flat_scores)
    shape = (BATCH, 2 * BEAM)
    return vals.reshape(shape), beams.reshape(shape), toks.reshape(shape)


# speedup vs baseline: 10.8917x; 1.1378x over previous
import functools

import jax
import jax.numpy as jnp
from jax import lax
from jax.experimental import pallas as pl
from jax.experimental.pallas import tpu as pltpu
from jax.experimental.pallas import tpu_sc as plsc

V = 100000
BATCH = 64
BEAM = 4
K = 8
NSW = 9
CW = 6144
NCHUNK = 8
REMW = 1664
SLABC_W = 1792
GMB = 6400
NEG = -1e30
IMAX = 2**31 - 1
LN2 = 0.6931471805599453


def _log_f32(v):
    bits = lax.bitcast_convert_type(v, jnp.int32)
    e = ((bits >> 23) & 0xFF) - 127
    mant = lax.bitcast_convert_type((bits & 0x7FFFFF) | 0x3F800000,
                                    jnp.float32)
    big = mant > 1.41421356
    mant = jnp.where(big, mant * 0.5, mant)
    ef = jnp.where(big, e + 1, e).astype(jnp.float32)
    t = (mant - 1.0) / (mant + 1.0)
    t2 = t * t
    p = 2.0 * t * (1.0 + t2 * (1.0 / 3.0 + t2 * (0.2 + t2 * (1.0 / 7.0))))
    return ef * LN2 + p


@functools.partial(
    pl.kernel,
    mesh=plsc.VectorSubcoreMesh(core_axis_name="c", subcore_axis_name="s"),
    compiler_params=pltpu.CompilerParams(needs_layout_passes=False),
    out_type=[
        jax.ShapeDtypeStruct((BATCH * 2 * BEAM,), jnp.float32),
        jax.ShapeDtypeStruct((BATCH * 2 * BEAM,), jnp.int32),
        jax.ShapeDtypeStruct((BATCH * 2 * BEAM,), jnp.int32),
    ],
    scratch_types=[
        pltpu.VMEM((BEAM, CW), jnp.float32),
        pltpu.VMEM((BEAM, CW), jnp.float32),
        pltpu.VMEM((BEAM, SLABC_W), jnp.float32),
        pltpu.VMEM((BEAM * 32,), jnp.float32),
        pltpu.VMEM((BEAM * GMB,), jnp.float32),
        pltpu.VMEM((BEAM * 16,), jnp.float32),
        pltpu.VMEM((16,), jnp.float32),
        pltpu.VMEM((BEAM * 16,), jnp.float32),
        pltpu.VMEM((BEAM * 16,), jnp.int32),
        pltpu.VMEM((BEAM * 16,), jnp.float32),
        pltpu.VMEM((BEAM * 16,), jnp.int32),
        pltpu.VMEM((NSW * BEAM, BEAM, 256), jnp.float32),
        pltpu.VMEM((16,), jnp.float32),
        pltpu.VMEM((16,), jnp.int32),
        pltpu.VMEM((16,), jnp.int32),
        pltpu.SemaphoreType.DMA,
        pltpu.SemaphoreType.DMA,
        pltpu.SemaphoreType.DMA,
        pltpu.SemaphoreType.DMA,
    ],
)
def _topk_sc(logits_hbm, tail_hbm, scores_hbm, vals_hbm, beam_hbm, tok_hbm,
             slaba, slabb, slabc, tailbuf, gmbuf, msbuf, scorebuf,
             resvals, residx, stats, winbuf, candbuf, stagef, stageb, staget,
             sema, semb, semr, semc):
    tid = lax.axis_index("s") * 2 + lax.axis_index("c")
    lane = lax.iota(jnp.int32, 16)
    negv = jnp.full((16,), NEG, jnp.float32)
    zf = jnp.zeros((16,), jnp.float32)
    zi = jnp.zeros((16,), jnp.int32)

    for m in range(BEAM):
        for k in range((GMB - 6256) // 16):
            gmbuf[pl.ds(m * GMB + 6256 + k * 16, 16)] = negv
        for k in range((SLABC_W - 1696) // 16):
            slabc[m, pl.ds(1696 + k * 16, 16)] = negv
    scorebuf[...] = zf
    pltpu.sync_copy(scores_hbm.at[pl.ds(tid * 8, 8)], scorebuf.at[pl.ds(0, 8)])

    def stream_pairs(src, p, pbase, svs):
        out = []
        base = pl.multiple_of(p * 256, 256)
        for m in range(BEAM):
            vs = [src[m, pl.ds(base + k * 16, 16)] for k in range(16)]
            t = vs
            for stride in (8, 4, 2, 1):
                t = [jnp.maximum(t[j], t[j + stride]) for j in range(stride)]
            gmbuf[pl.ds(m * GMB + (pbase + p) * 16, 16)] = t[0]
            es = [jnp.exp(v) for v in vs]
            for stride in (8, 4, 2, 1):
                es = [es[j] + es[j + stride] for j in range(stride)]
            out.append(svs[m] + es[0])
        return tuple(out)

    def batch_body(bl, _carry):
        b = tid * 2 + bl
        pltpu.async_copy(logits_hbm.at[b, :, pl.ds(0, CW)], slaba, sema)
        pltpu.async_copy(logits_hbm.at[b, :, pl.ds(16 * CW, REMW)],
                         slabc.at[:, pl.ds(0, REMW)], semr)
        pltpu.async_copy(tail_hbm.at[pl.ds(b * 128, 128)], tailbuf, semr)

        def chunk_body(i, svs):
            c0 = pl.multiple_of((2 * i + 1) * CW, CW)
            pltpu.async_copy(logits_hbm.at[b, :, pl.ds(c0, CW)], slabb, semb)
            pltpu.make_async_copy(logits_hbm.at[b, :, pl.ds(0, CW)],
                                  slaba, sema).wait()

            def pair_a(p, svsi):
                return stream_pairs(slaba, p, i * 48, svsi)

            svs = lax.fori_loop(0, 24, pair_a, svs)

            @pl.when(i < NCHUNK - 1)
            def _():
                c1 = pl.multiple_of((2 * i + 2) * CW, CW)
                pltpu.async_copy(logits_hbm.at[b, :, pl.ds(c1, CW)],
                                 slaba, sema)

            pltpu.make_async_copy(logits_hbm.at[b, :, pl.ds(0, CW)],
                                  slabb, semb).wait()

            def pair_b(p, svsi):
                return stream_pairs(slabb, p, i * 48 + 24, svsi)

            return lax.fori_loop(0, 24, pair_b, svs)

        svs = lax.fori_loop(0, NCHUNK, chunk_body, (zf, zf, zf, zf))

        pltpu.make_async_copy(logits_hbm.at[b, :, pl.ds(0, REMW)],
                              slabc.at[:, pl.ds(0, REMW)], semr).wait()
        pltpu.make_async_copy(tail_hbm.at[pl.ds(0, 128)], tailbuf,
                              semr).wait()
        for m in range(BEAM):
            slabc[m, pl.ds(1664, 16)] = tailbuf[pl.ds(m * 32, 16)]
            slabc[m, pl.ds(1680, 16)] = tailbuf[pl.ds(m * 32 + 16, 16)]

        def pair_c(p, svsi):
            return stream_pairs(slabc, p, 384, svsi)

        svs = lax.fori_loop(0, 7, pair_c, svs)
        for m in range(BEAM):
            msbuf[pl.ds(m * 16, 16)] = svs[m]

        def beam_sel(m, _c):
            sv = msbuf[pl.ds(m * 16, 16)]
            s = jnp.sum(sv)
            lse = jnp.max(_log_f32(jnp.full((16,), s, jnp.float32)))
            sc = jnp.max(jnp.where(lane == bl * BEAM + m, scorebuf[...], NEG))
            adj = sc - lse

            l2 = []
            for cl in range(25):
                t = [gmbuf[pl.ds(m * GMB + cl * 256 + k * 16, 16)]
                     for k in range(16)]
                for stride in (8, 4, 2, 1):
                    t = [jnp.maximum(t[j], t[j + stride])
                         for j in range(stride)]
                l2.append(t[0])

            wv = zi
            for sw in range(NSW):
                val_v, g2v = negv, zi
                for cl in range(25):
                    upd = l2[cl] > val_v
                    val_v = jnp.where(upd, l2[cl], val_v)
                    g2v = jnp.where(upd, cl * 16 + lane, g2v)
                ml2 = jnp.max(val_v)
                g2 = jnp.min(jnp.where(val_v == ml2, g2v, IMAX))
                clsel, l2l = g2 >> 4, g2 & 15
                idxrel = clsel * 256 + l2l + 16 * lane
                gmv = plsc.load_gather(gmbuf, [m * GMB + idxrel])
                g = jnp.min(jnp.where(gmv == ml2, idxrel, IMAX))
                plsc.store_scatter(gmbuf,
                                   [jnp.full((16,), m * GMB + g, jnp.int32)],
                                   negv, mask=lane == 0)
                newl2 = jnp.max(jnp.where(idxrel == g, NEG, gmv))
                hit = (lane == l2l) & jnp.full((16,), True)
                l2 = [jnp.where((cl == clsel) & hit, newl2, l2[cl])
                      for cl in range(25)]
                wv = jnp.where(lane == sw, g, wv)

                p = g >> 4
                srccol = pl.multiple_of(jnp.where(p < 390, p, 0) * 256, 256)
                pltpu.async_copy(logits_hbm.at[b, :, pl.ds(srccol, 256)],
                                 candbuf.at[m * NSW + sw], semc)

            winbuf[pl.ds(m * 16, 16)] = wv
            resvals[pl.ds(pl.multiple_of(m * 16, 16), 16)] = negv
            stats[pl.ds(m * 16, 16)] = jnp.where(lane == 0, adj, 0.0)
            return 0

        lax.fori_loop(0, BEAM, beam_sel, 0)
        for _w in range(NSW * BEAM):
            pltpu.make_async_copy(logits_hbm.at[b, :, pl.ds(0, 256)],
                                  candbuf.at[0], semc).wait()

        def beam_top(m, _c):
            wv = winbuf[pl.ds(m * 16, 16)]
            mfull = jnp.full((16,), m, jnp.int32)
            cand_v, cand_c = [], []
            for w in range(NSW):
                g = jnp.max(jnp.where(lane == w, wv, -1))
                p, l = g >> 4, g & 15
                colv = p * 256 + l + 16 * lane
                va = plsc.load_gather(
                    candbuf, [jnp.full((16,), m * NSW + w, jnp.int32), mfull,
                              l + 16 * lane])
                vb = plsc.load_gather(slabc, [mfull, 1536 + l + 16 * lane])
                val = jnp.where(p == 390, vb, va)
                val = jnp.where((colv >= V) | (colv == 1), NEG, val)
                cand_v.append(val)
                cand_c.append(colv)

            v8, i8 = negv, zi
            for j in range(K):
                mx = cand_v[0]
                for q in range(1, NSW):
                    mx = jnp.maximum(mx, cand_v[q])
                mc = jnp.max(mx)
                cur = jnp.full((16,), IMAX, jnp.int32)
                for q in range(NSW):
                    cur = jnp.minimum(cur, jnp.where(cand_v[q] == mc,
                                                     cand_c[q], IMAX))
                chosen = jnp.min(cur)
                v8 = jnp.where(lane == j, mc, v8)
                i8 = jnp.where(lane == j, chosen, i8)
                cand_v = [jnp.where(cand_c[q] == chosen, NEG, cand_v[q])
                          for q in range(NSW)]

            resvals[pl.ds(m * 16, 16)] = v8
            residx[pl.ds(m * 16, 16)] = i8
            return 0

        lax.fori_loop(0, BEAM, beam_top, 0)

        comb, flat = [], []
        for m in range(BEAM):
            svv = stats[pl.ds(m * 16, 16)]
            adj = jnp.max(jnp.where(lane == 0, svv, NEG))
            comb.append(jnp.where(lane < K,
                                  resvals[pl.ds(m * 16, 16)] + adj, NEG))
            flat.append(residx[pl.ds(m * 16, 16)] + m * V)
        ov, ob, ot = negv, zi, zi
        for j in range(K):
            mx = jnp.maximum(jnp.maximum(comb[0], comb[1]),
                             jnp.maximum(comb[2], comb[3]))
            mc = jnp.max(mx)
            cur = jnp.full((16,), IMAX, jnp.int32)
            for m in range(BEAM):
                cur = jnp.minimum(cur, jnp.where(comb[m] == mc,
                                                 flat[m], IMAX))
            fm = jnp.min(cur)
            ov = jnp.where(lane == j, mc, ov)
            ob = jnp.where(lane == j, fm // V, ob)
            ot = jnp.where(lane == j, fm % V, ot)
            comb = [jnp.where(flat[m] == fm, NEG, comb[m])
                    for m in range(BEAM)]
        stagef[...] = ov
        stageb[...] = ob
        staget[...] = ot
        obase = pl.multiple_of(b * K, 8)
        pltpu.sync_copy(stagef.at[pl.ds(0, K)], vals_hbm.at[pl.ds(obase, K)])
        pltpu.sync_copy(stageb.at[pl.ds(0, K)], beam_hbm.at[pl.ds(obase, K)])
        pltpu.sync_copy(staget.at[pl.ds(0, K)], tok_hbm.at[pl.ds(obase, K)])
        return 0

    lax.fori_loop(0, 2, batch_body, 0)


def kernel(logits, scores):
    tail = logits[:, :, V - 32:].reshape(BATCH * BEAM * 32)
    flat_scores = scores.reshape(BATCH * BEAM)
    vals, beams, toks = _topk_sc(logits, tail, flat_scores)
    shape = (BATCH, 2 * BEAM)
    return vals.reshape(shape), beams.reshape(shape), toks.reshape(shape)
